# baseline (speedup 1.0000x reference)
import functools

import jax
import jax.numpy as jnp
from jax import lax
from jax.experimental import pallas as pl
from jax.experimental.pallas import tpu as pltpu
from jax.experimental.pallas import tpu_sc as plsc

D = 64
SCALE = 8.0
NC = 2
NS = 16
NW = NC * NS
B_TOTAL = 4096 * 200
B_PER_W = B_TOTAL // NW
GATHER = 128
KSUB = 4
CHUNK = GATHER * KSUB
NCHUNK = B_PER_W // CHUNK
IDX_ROWS_PER_W = B_PER_W // GATHER

_mesh = plsc.VectorSubcoreMesh(core_axis_name="c", subcore_axis_name="s")


@functools.partial(
    pl.kernel,
    mesh=_mesh,
    out_type=jax.ShapeDtypeStruct((B_TOTAL, D), jnp.float32),
    scratch_types=[
        pltpu.VMEM((2, KSUB, GATHER), jnp.int32),
        pltpu.VMEM((2, CHUNK, D), jnp.float32),
        pltpu.SemaphoreType.DMA,
        pltpu.SemaphoreType.DMA,
    ],
    compiler_params=pltpu.CompilerParams(use_tc_tiling_on_sc=False),
)
def _emb_lookup(idx_hbm, table_hbm, out_hbm, idx_v, rows_v, g0, g1):
    wid = lax.axis_index("s") * NC + lax.axis_index("c")
    base = wid * B_PER_W
    idx_base = wid * IDX_ROWS_PER_W
    gsems = (g0, g1)

    def fire(i, b):
        pltpu.sync_copy(idx_hbm.at[pl.ds(idx_base + i * KSUB, KSUB)],
                        idx_v.at[b])
        for j in range(KSUB):
            pltpu.async_copy(table_hbm.at[idx_v.at[b, j]],
                             rows_v.at[b, pl.ds(j * GATHER, GATHER)],
                             gsems[b])

    def process(i, b):
        pltpu.make_async_copy(out_hbm.at[pl.ds(base, CHUNK)],
                              rows_v.at[b], gsems[b]).wait()

        def scale_row(r, carry):
            for c in range(D // 16):
                sl = pl.ds(c * 16, 16)
                rows_v[b, r, sl] = rows_v[b, r, sl] * SCALE
            return carry

        lax.fori_loop(0, CHUNK, scale_row, 0)
        pltpu.sync_copy(rows_v.at[b], out_hbm.at[pl.ds(base + i * CHUNK, CHUNK)])

    fire(0, 0)

    def outer(t, carry):
        i0 = 2 * t
        fire(i0 + 1, 1)
        process(i0, 0)

        @pl.when(t + 1 < NCHUNK // 2)
        def _():
            fire(i0 + 2, 0)

        process(i0 + 1, 1)
        return carry

    lax.fori_loop(0, NCHUNK // 2, outer, 0)


def kernel(x, table):
    x_flat = x.astype(jnp.int32).reshape(B_TOTAL // GATHER, GATHER)
    out = _emb_lookup(x_flat, table)
    return out.reshape(x.shape[0], x.shape[1], D)
